# baseline (speedup 1.0000x reference)
import functools

import jax
import jax.numpy as jnp
import numpy as np
from jax import lax
from jax.experimental import pallas as pl
from jax.experimental.pallas import tpu as pltpu
from jax.experimental.pallas import tpu_sc as plsc

N = 10000
E = 320000
D = 128
F = 128

NC = 2
NS = 16
L = 16
FH = F // NC
NCH = F // FH

EB = 80
EPT = E // NS
NB1 = EPT // EB
NW = NC * NS
EPW = E // NW
NB2 = EPW // EB

NPAD = 10240
RPT = NPAD // NS
LAST_ROWS = N - (NS - 1) * RPT
NODE_BLK = 1000

_PCORE = np.concatenate([
    np.concatenate([np.arange(0, 32, 2), np.arange(1, 32, 2)]) + 32 * b
    for b in range(FH // 32)
])
_PFULL = np.concatenate([_PCORE, _PCORE + FH])


def _ab_body(x_ref, wa0_ref, wa1_ref, wb0_ref, wb1_ref, a_ref, b_ref):
    xb = x_ref[...]
    a_ref[0] = jnp.dot(
        xb, wa0_ref[...], preferred_element_type=jnp.float32
    ).astype(jnp.bfloat16)
    a_ref[1] = jnp.dot(
        xb, wa1_ref[...], preferred_element_type=jnp.float32
    ).astype(jnp.bfloat16)
    b_ref[0] = jnp.dot(
        xb, wb0_ref[...], preferred_element_type=jnp.float32
    ).astype(jnp.bfloat16)
    b_ref[1] = jnp.dot(
        xb, wb1_ref[...], preferred_element_type=jnp.float32
    ).astype(jnp.bfloat16)


_ab_call = pl.pallas_call(
    _ab_body,
    grid=(N // NODE_BLK,),
    in_specs=[
        pl.BlockSpec((NODE_BLK, D), lambda i: (i, 0)),
        pl.BlockSpec((D, FH), lambda i: (0, 0)),
        pl.BlockSpec((D, FH), lambda i: (0, 0)),
        pl.BlockSpec((D, FH), lambda i: (0, 0)),
        pl.BlockSpec((D, FH), lambda i: (0, 0)),
    ],
    out_specs=[
        pl.BlockSpec((NC, NODE_BLK, FH), lambda i: (0, i, 0)),
        pl.BlockSpec((NC, NODE_BLK, FH), lambda i: (0, i, 0)),
    ],
    out_shape=[
        jax.ShapeDtypeStruct((NC, N, FH), jnp.bfloat16),
        jax.ShapeDtypeStruct((NC, N, FH), jnp.bfloat16),
    ],
)

_C_BLK = 8000


def _c_body(ea_ref, wc0_ref, wc1_ref, eb1a_ref, eb1b_ref, c_ref):
    ea = ea_ref[...]
    c_ref[0] = (
        jnp.dot(ea, wc0_ref[...], preferred_element_type=jnp.float32)
        + eb1a_ref[...]
    ).astype(jnp.bfloat16)
    c_ref[1] = (
        jnp.dot(ea, wc1_ref[...], preferred_element_type=jnp.float32)
        + eb1b_ref[...]
    ).astype(jnp.bfloat16)


_c_call = pl.pallas_call(
    _c_body,
    grid=(E // _C_BLK,),
    in_specs=[
        pl.BlockSpec((_C_BLK, 5), lambda i: (i, 0)),
        pl.BlockSpec((5, FH), lambda i: (0, 0)),
        pl.BlockSpec((5, FH), lambda i: (0, 0)),
        pl.BlockSpec((1, FH), lambda i: (0, 0)),
        pl.BlockSpec((1, FH), lambda i: (0, 0)),
    ],
    out_specs=pl.BlockSpec((NC, _C_BLK, FH), lambda i: (0, i, 0)),
    out_shape=jax.ShapeDtypeStruct((NC, E, FH), jnp.bfloat16),
)


def _node_body(x_ref, h0_ref, h1_ref, dg_ref, cr0_ref, cr1_ref, cc0_ref,
               cc1_ref, pos_ref, ew2_ref, eb2_ref, nw1x_ref, nw1a_ref,
               nb1_ref, nw2_ref, nb2_ref, out_ref, co_ref):
    hs = jnp.concatenate([h0_ref[0], h1_ref[0]], axis=1)
    deg = dg_ref[...][:, 0:1]
    agg = (
        jnp.dot(hs, ew2_ref[...], preferred_element_type=jnp.float32)
        + deg * eb2_ref[...]
    )
    z = (
        jnp.dot(x_ref[...], nw1x_ref[...], preferred_element_type=jnp.float32)
        + jnp.dot(agg, nw1a_ref[...], preferred_element_type=jnp.float32)
        + nb1_ref[...]
    )
    h2 = jnp.maximum(z, 0.0)
    out_ref[...] = (
        jnp.dot(h2, nw2_ref[...], preferred_element_type=jnp.float32)
        + nb2_ref[...]
    )
    cw = (cr0_ref[0][:, 0:1] + cr1_ref[0][:, 0:1]
          - cc0_ref[0][:, 0:1] - cc1_ref[0][:, 0:1])
    co_ref[...] = jnp.sum(cw * pos_ref[...], axis=0, keepdims=True)[None]


_node_call = pl.pallas_call(
    _node_body,
    grid=(N // NODE_BLK,),
    in_specs=[
        pl.BlockSpec((NODE_BLK, D), lambda i: (i, 0)),
        pl.BlockSpec((1, NODE_BLK, FH), lambda i: (0, i, 0)),
        pl.BlockSpec((1, NODE_BLK, FH), lambda i: (1, i, 0)),
        pl.BlockSpec((NODE_BLK, L), lambda i: (i, 0)),
        pl.BlockSpec((1, NODE_BLK, L), lambda i: (0, i, 0)),
        pl.BlockSpec((1, NODE_BLK, L), lambda i: (1, i, 0)),
        pl.BlockSpec((1, NODE_BLK, L), lambda i: (0, i, 0)),
        pl.BlockSpec((1, NODE_BLK, L), lambda i: (1, i, 0)),
        pl.BlockSpec((NODE_BLK, 3), lambda i: (i, 0)),
        pl.BlockSpec((F, F), lambda i: (0, 0)),
        pl.BlockSpec((1, F), lambda i: (0, 0)),
        pl.BlockSpec((D, F), lambda i: (0, 0)),
        pl.BlockSpec((F, F), lambda i: (0, 0)),
        pl.BlockSpec((1, F), lambda i: (0, 0)),
        pl.BlockSpec((F, F), lambda i: (0, 0)),
        pl.BlockSpec((1, F), lambda i: (0, 0)),
    ],
    out_specs=[
        pl.BlockSpec((NODE_BLK, F), lambda i: (i, 0)),
        pl.BlockSpec((1, 1, 3), lambda i: (i, 0, 0)),
    ],
    out_shape=[
        jax.ShapeDtypeStruct((N, F), jnp.float32),
        jax.ShapeDtypeStruct((N // NODE_BLK, 1, 3), jnp.float32),
    ],
)


_mesh = plsc.VectorSubcoreMesh(
    core_axis_name="c", subcore_axis_name="s", num_cores=NC, num_subcores=NS)

_sc_params = pltpu.CompilerParams(
    needs_layout_passes=False, use_tc_tiling_on_sc=False)


@functools.partial(
    pl.kernel,
    out_type=[
        jax.ShapeDtypeStruct((NC, N, FH), jnp.float32),
        jax.ShapeDtypeStruct((N, L), jnp.float32),
        jax.ShapeDtypeStruct((NC * E,), jnp.float32),
    ],
    mesh=_mesh,
    compiler_params=_sc_params,
    scratch_types=[
        pltpu.VMEM((2, EB), jnp.int32),
        pltpu.VMEM((2, EB), jnp.int32),
        pltpu.VMEM((2, EB), jnp.int32),
        pltpu.VMEM((2, EB), jnp.int32),
        pltpu.VMEM((2, EB), jnp.int32),
        pltpu.VMEM((2, EB, FH), jnp.bfloat16),
        pltpu.VMEM((2, EB, FH), jnp.bfloat16),
        pltpu.VMEM((2, EB, FH), jnp.bfloat16),
        pltpu.VMEM((2, EB, FH), jnp.float32),
        pltpu.VMEM((2, EB), jnp.float32),
        pltpu.VMEM((EB, L), jnp.float32),
        pltpu.VMEM((EB, L), jnp.float32),
        pltpu.VMEM((FH,), jnp.float32),
        pltpu.VMEM_SHARED((NPAD, FH), jnp.float32),
        pltpu.VMEM_SHARED((NPAD, L), jnp.float32),
        pltpu.SemaphoreType.DMA,
        pltpu.SemaphoreType.DMA,
        pltpu.SemaphoreType.DMA,
        pltpu.SemaphoreType.DMA,
        pltpu.SemaphoreType.DMA,
        pltpu.SemaphoreType.DMA,
        pltpu.SemaphoreType.DMA,
        pltpu.SemaphoreType.DMA,
        pltpu.SemaphoreType.DMA,
        pltpu.SemaphoreType.DMA,
        pltpu.SemaphoreType.DMA,
        pltpu.SemaphoreType.DMA,
    ],
)
def _edge_call(rowi_hbm, coli_hbm, a_hbm, b_hbm, c_hbm, w_hbm,
               hs_out, dg_out, s_out,
               rowb_v, colb_v, row2b_v, col2b_v, rscb_v, arb_v, bcb_v, cb_v,
               hb_v, sb_v, ones_v, sacc_v, w_v, hsh, dsh,
               semga0, semga1, semgb0, semgb1, semgc0, semgc1,
               semh0, semh1, semd0, semd1, sems0, sems1):
    cid = lax.axis_index("c")
    sid = lax.axis_index("s")

    pltpu.sync_copy(w_hbm.at[cid], w_v)

    zero16 = jnp.zeros((L,), jnp.float32)
    idx16 = lax.iota(jnp.int32, L)
    onevec = jnp.where(idx16 == 0, 1.0, 0.0).astype(jnp.float32)
    semga = [semga0, semga1]
    semgb = [semgb0, semgb1]
    semgc = [semgc0, semgc1]
    semh = [semh0, semh1]
    semd = [semd0, semd1]
    sems = [sems0, sems1]

    def _zrow(j, carry):
        for kk in range(FH // L):
            hb_v[0, j, pl.ds(kk * L, L)] = zero16
        ones_v[j, :] = zero16
        return carry

    lax.fori_loop(0, EB, _zrow, 0)

    def _zcp(t, carry):
        r0 = sid * RPT + t * EB
        pltpu.sync_copy(hb_v.at[0], hsh.at[pl.ds(r0, EB)])
        pltpu.sync_copy(ones_v, dsh.at[pl.ds(r0, EB)])
        return carry

    lax.fori_loop(0, RPT // EB, _zcp, 0)

    def _orow(j, carry):
        ones_v[j, :] = onevec
        return carry

    lax.fori_loop(0, EB, _orow, 0)

    plsc.subcore_barrier()

    wregs = [w_v[pl.ds(kk * L, L)] for kk in range(FH // L)]
    roff = cid * N
    ebase = sid * EPT

    def _issue(p, bi):
        base = ebase + bi * EB
        pltpu.sync_copy(rowi_hbm.at[pl.ds(base, EB)], rowb_v.at[p])
        pltpu.sync_copy(coli_hbm.at[pl.ds(base, EB)], colb_v.at[p])
        for g in range(EB // L):
            s = pl.ds(g * L, L)
            row2b_v[p, s] = rowb_v[p, s] + roff
            col2b_v[p, s] = colb_v[p, s] + roff
        pltpu.async_copy(a_hbm.at[row2b_v.at[p]], arb_v.at[p], semga[p])
        pltpu.async_copy(b_hbm.at[col2b_v.at[p]], bcb_v.at[p], semgb[p])
        pltpu.async_copy(c_hbm.at[pl.ds(cid * E + base, EB)], cb_v.at[p],
                         semgc[p])

    def _waitg(p, bi):
        base = ebase + bi * EB
        pltpu.make_async_copy(
            a_hbm.at[row2b_v.at[p]], arb_v.at[p], semga[p]).wait()
        pltpu.make_async_copy(
            b_hbm.at[col2b_v.at[p]], bcb_v.at[p], semgb[p]).wait()
        pltpu.make_async_copy(
            c_hbm.at[pl.ds(cid * E + base, EB)], cb_v.at[p], semgc[p]).wait()

    def _compute(p, bi):
        def _row(j, rc):
            acc = zero16
            for kk in range(FH // (2 * L)):
                s32 = pl.ds(kk * 2 * L, 2 * L)
                ae, ao = plsc.unpack(
                    arb_v[p, j, s32], format=plsc.PackFormat.INTERLEAVED)
                be, bo = plsc.unpack(
                    bcb_v[p, j, s32], format=plsc.PackFormat.INTERLEAVED)
                ce, co = plsc.unpack(
                    cb_v[p, j, s32], format=plsc.PackFormat.INTERLEAVED)
                he = jnp.maximum(ae + be + ce, 0.0)
                ho = jnp.maximum(ao + bo + co, 0.0)
                hb_v[p, j, pl.ds(kk * 2 * L, L)] = he
                hb_v[p, j, pl.ds(kk * 2 * L + L, L)] = ho
                acc = acc + he * wregs[2 * kk] + ho * wregs[2 * kk + 1]
            sacc_v[j, :] = acc
            return rc

        lax.fori_loop(0, EB, _row, 0)

        for g in range(EB // L):
            eidx = idx16 + g * L
            sv = zero16
            for l in range(L):
                sv = sv + plsc.load_gather(
                    sacc_v, [eidx, jnp.full((L,), l, jnp.int32)])
            sb_v[p, pl.ds(g * L, L)] = sv
            s = pl.ds(g * L, L)
            rscb_v[p, s] = rowb_v[p, s]

    def _scatter(p, bi):
        base = ebase + bi * EB
        pltpu.async_copy(hb_v.at[p], hsh.at[rscb_v.at[p]], semh[p],
                         add=True)

        @pl.when(cid == 0)
        def _deg():
            pltpu.async_copy(ones_v, dsh.at[rscb_v.at[p]], semd[p],
                             add=True)

        pltpu.async_copy(sb_v.at[p], s_out.at[pl.ds(cid * E + base, EB)],
                         sems[p])

    def _drain_sc(p, bi_prev):
        base = ebase + bi_prev * EB
        pltpu.make_async_copy(
            hb_v.at[p], hsh.at[rscb_v.at[p]], semh[p]).wait()

        @pl.when(cid == 0)
        def _deg():
            pltpu.make_async_copy(
                ones_v, dsh.at[rscb_v.at[p]], semd[p]).wait()

        pltpu.make_async_copy(
            sb_v.at[p], s_out.at[pl.ds(cid * E + base, EB)], sems[p]).wait()

    NHALF = NB1 // 2
    _issue(0, 0)
    _issue(1, 1)
    _waitg(0, 0)
    _compute(0, 0)
    _scatter(0, 0)
    _issue(0, 2)
    _waitg(1, 1)
    _compute(1, 1)
    _scatter(1, 1)
    _issue(1, 3)

    def _pair(k, carry):
        for p in range(2):
            bi = 2 * k + p
            _drain_sc(p, bi - 2)
            _waitg(p, bi)
            _compute(p, bi)
            _scatter(p, bi)

            @pl.when(k < NHALF - 1)
            def _next():
                _issue(p, bi + 2)

        return carry

    lax.fori_loop(1, NHALF, _pair, 0)
    _drain_sc(0, NB1 - 2)
    _drain_sc(1, NB1 - 1)

    plsc.subcore_barrier()

    r0 = sid * RPT

    @pl.when(sid < NS - 1)
    def _drain_full():
        pltpu.sync_copy(hsh.at[pl.ds(r0, RPT)], hs_out.at[cid, pl.ds(r0, RPT)])

        @pl.when(cid == 0)
        def _():
            pltpu.sync_copy(dsh.at[pl.ds(r0, RPT)], dg_out.at[pl.ds(r0, RPT)])

    @pl.when(sid == NS - 1)
    def _drain_last():
        pltpu.sync_copy(hsh.at[pl.ds(r0, LAST_ROWS)],
                        hs_out.at[cid, pl.ds(r0, LAST_ROWS)])

        @pl.when(cid == 0)
        def _():
            pltpu.sync_copy(dsh.at[pl.ds(r0, LAST_ROWS)],
                            dg_out.at[pl.ds(r0, LAST_ROWS)])


@functools.partial(
    pl.kernel,
    out_type=[
        jax.ShapeDtypeStruct((NC, N, L), jnp.float32),
        jax.ShapeDtypeStruct((NC, N, L), jnp.float32),
    ],
    mesh=_mesh,
    compiler_params=_sc_params,
    scratch_types=[
        pltpu.VMEM((2, EB), jnp.int32),
        pltpu.VMEM((2, EB), jnp.int32),
        pltpu.VMEM((2, EB), jnp.int32),
        pltpu.VMEM((2, EB), jnp.int32),
        pltpu.VMEM((2, EB), jnp.float32),
        pltpu.VMEM((2, EB), jnp.float32),
        pltpu.VMEM((2, EB, L), jnp.float32),
        pltpu.VMEM((L,), jnp.float32),
        pltpu.VMEM_SHARED((NPAD, L), jnp.float32),
        pltpu.VMEM_SHARED((NPAD, L), jnp.float32),
        pltpu.SemaphoreType.DMA,
        pltpu.SemaphoreType.DMA,
        pltpu.SemaphoreType.DMA,
        pltpu.SemaphoreType.DMA,
        pltpu.SemaphoreType.DMA,
        pltpu.SemaphoreType.DMA,
        pltpu.SemaphoreType.DMA,
        pltpu.SemaphoreType.DMA,
    ],
)
def _coord_call(rowi_hbm, coli_hbm, s_hbm, k_hbm, cr_out, cc_out,
                rowb_v, colb_v, rscb_v, cscb_v, s0b_v, s1b_v, cbufb_v, k_v,
                crsh, ccsh,
                semi0, semi1, sems0, sems1, semr0, semr1, semc0, semc1):
    cid = lax.axis_index("c")
    sid = lax.axis_index("s")
    wid = sid * NC + cid

    pltpu.sync_copy(k_hbm, k_v)
    zero16 = jnp.zeros((L,), jnp.float32)
    idx16 = lax.iota(jnp.int32, L)
    zeroi = jnp.zeros((L,), jnp.int32)
    semi = [semi0, semi1]
    semsv = [sems0, sems1]
    semr = [semr0, semr1]
    semc = [semc0, semc1]

    def _zrow(j, carry):
        for p in range(2):
            cbufb_v[p, j, :] = zero16
        return carry

    lax.fori_loop(0, EB, _zrow, 0)

    def _zcp(t, carry):
        r0 = sid * RPT + t * EB
        pltpu.sync_copy(cbufb_v.at[0], crsh.at[pl.ds(r0, EB)])
        pltpu.sync_copy(cbufb_v.at[0], ccsh.at[pl.ds(r0, EB)])
        return carry

    lax.fori_loop(0, RPT // EB, _zcp, 0)

    plsc.subcore_barrier()

    kvec = k_v[:]
    b0 = kvec[0]
    cw2s = kvec[1]
    cb2s = kvec[2]
    ebase = wid * EPW

    def _issue(p, bi):
        base = ebase + bi * EB
        pltpu.async_copy(rowi_hbm.at[pl.ds(base, EB)], rowb_v.at[p], semi[p])
        pltpu.async_copy(coli_hbm.at[pl.ds(base, EB)], colb_v.at[p], semi[p])
        pltpu.async_copy(s_hbm.at[pl.ds(base, EB)], s0b_v.at[p], semsv[p])
        pltpu.async_copy(s_hbm.at[pl.ds(E + base, EB)], s1b_v.at[p], semsv[p])

    def _waitin(p, bi):
        base = ebase + bi * EB
        pltpu.make_async_copy(
            rowi_hbm.at[pl.ds(base, EB)], rowb_v.at[p], semi[p]).wait()
        pltpu.make_async_copy(
            coli_hbm.at[pl.ds(base, EB)], colb_v.at[p], semi[p]).wait()
        pltpu.make_async_copy(
            s_hbm.at[pl.ds(base, EB)], s0b_v.at[p], semsv[p]).wait()
        pltpu.make_async_copy(
            s_hbm.at[pl.ds(E + base, EB)], s1b_v.at[p], semsv[p]).wait()

    def _compute(p):
        for g in range(EB // L):
            s = pl.ds(g * L, L)
            sv = s0b_v[p, s] + s1b_v[p, s]
            cvec = jnp.maximum(sv + b0, 0.0) * cw2s + cb2s
            plsc.store_scatter(cbufb_v.at[p], [idx16 + g * L, zeroi], cvec)
            rscb_v[p, s] = rowb_v[p, s]
            cscb_v[p, s] = colb_v[p, s]

    def _scatter(p):
        pltpu.async_copy(cbufb_v.at[p], crsh.at[rscb_v.at[p]], semr[p],
                         add=True)
        pltpu.async_copy(cbufb_v.at[p], ccsh.at[cscb_v.at[p]], semc[p],
                         add=True)

    def _drain(p):
        pltpu.make_async_copy(
            cbufb_v.at[p], crsh.at[rscb_v.at[p]], semr[p]).wait()
        pltpu.make_async_copy(
            cbufb_v.at[p], ccsh.at[cscb_v.at[p]], semc[p]).wait()

    NHALF2 = NB2 // 2
    _issue(0, 0)
    _issue(1, 1)
    _waitin(0, 0)
    _compute(0)
    _scatter(0)
    _issue(0, 2)
    _waitin(1, 1)
    _compute(1)
    _scatter(1)
    _issue(1, 3)

    def _pair(k, carry):
        for p in range(2):
            bi = 2 * k + p
            _drain(p)
            _waitin(p, bi)
            _compute(p)
            _scatter(p)

            @pl.when(bi + 2 < NB2)
            def _next():
                _issue(p, bi + 2)

        return carry

    lax.fori_loop(1, NHALF2, _pair, 0)
    if NB2 % 2 == 1:
        _drain(0)
        _waitin(0, NB2 - 1)
        _compute(0)
        _scatter(0)
    _drain(0)
    _drain(1)

    plsc.subcore_barrier()

    r0 = sid * RPT

    @pl.when(sid < NS - 1)
    def _drain_full():
        pltpu.sync_copy(crsh.at[pl.ds(r0, RPT)], cr_out.at[cid, pl.ds(r0, RPT)])
        pltpu.sync_copy(ccsh.at[pl.ds(r0, RPT)], cc_out.at[cid, pl.ds(r0, RPT)])

    @pl.when(sid == NS - 1)
    def _drain_last():
        pltpu.sync_copy(crsh.at[pl.ds(r0, LAST_ROWS)],
                        cr_out.at[cid, pl.ds(r0, LAST_ROWS)])
        pltpu.sync_copy(ccsh.at[pl.ds(r0, LAST_ROWS)],
                        cc_out.at[cid, pl.ds(r0, LAST_ROWS)])


def kernel(x, edge_index, edge_attr, pos,
           ew1, eb1, ew2, eb2,
           nw1, nb1, nw2, nb2,
           cw1, cb1, cw2, cb2):
    rowi = edge_index[0].astype(jnp.int32)
    coli = edge_index[1].astype(jnp.int32)

    a_st, b_st = _ab_call(
        x, ew1[:D, :FH], ew1[:D, FH:], ew1[D:2 * D, :FH], ew1[D:2 * D, FH:])
    c_st = _c_call(
        edge_attr, ew1[2 * D:, :FH], ew1[2 * D:, FH:],
        eb1[:FH].reshape(1, FH), eb1[FH:].reshape(1, FH))

    w = (ew2 @ cw1)[:, 0]
    b0 = eb2 @ cw1[:, 0] + cb1[0]
    consts = jnp.concatenate([
        jnp.reshape(b0, (1,)), jnp.reshape(cw2, (1,)), jnp.reshape(cb2, (1,)),
        jnp.zeros((L - 3,), jnp.float32),
    ])

    hs, dg, s_parts = _edge_call(
        rowi, coli,
        a_st.reshape(NC * N, FH), b_st.reshape(NC * N, FH),
        c_st.reshape(NC * E, FH), w[_PFULL].reshape(NC, FH))

    crow, ccol = _coord_call(rowi, coli, s_parts, consts)

    node_features, co_parts = _node_call(
        x, hs, hs, dg, crow, crow, ccol, ccol, pos, ew2[_PFULL],
        eb2.reshape(1, F),
        nw1[:D], nw1[D:], nb1.reshape(1, F), nw2, nb2.reshape(1, F))

    coord = co_parts.reshape(N // NODE_BLK, 3).sum(axis=0)
    pos_out = pos + coord[None, :]
    return node_features, pos_out


# speedup vs baseline: 1.1016x; 1.1016x over previous
import functools

import jax
import jax.numpy as jnp
import numpy as np
from jax import lax
from jax.experimental import pallas as pl
from jax.experimental.pallas import tpu as pltpu
from jax.experimental.pallas import tpu_sc as plsc

N = 10000
E = 320000
D = 128
F = 128

NC = 2
NS = 16
L = 16
FH = F // NC
NCH = F // FH

EB = 80
EPT = E // NS
NB1 = EPT // EB
NW = NC * NS
EPW = E // NW
NB2 = EPW // EB

NPAD = 10240
RPT = NPAD // NS
LAST_ROWS = N - (NS - 1) * RPT
NODE_BLK = 1000

_PCORE = np.concatenate([
    np.concatenate([np.arange(0, 32, 2), np.arange(1, 32, 2)]) + 32 * b
    for b in range(FH // 32)
])
_PFULL = np.concatenate([_PCORE, _PCORE + FH])


def _ab_body(x_ref, wa0_ref, wa1_ref, wb0_ref, wb1_ref, a_ref, b_ref):
    xb = x_ref[...]
    a_ref[0] = jnp.dot(xb, wa0_ref[...], preferred_element_type=jnp.float32)
    a_ref[1] = jnp.dot(xb, wa1_ref[...], preferred_element_type=jnp.float32)
    b_ref[0] = jnp.dot(xb, wb0_ref[...], preferred_element_type=jnp.float32)
    b_ref[1] = jnp.dot(xb, wb1_ref[...], preferred_element_type=jnp.float32)


_ab_call = pl.pallas_call(
    _ab_body,
    grid=(N // NODE_BLK,),
    in_specs=[
        pl.BlockSpec((NODE_BLK, D), lambda i: (i, 0)),
        pl.BlockSpec((D, FH), lambda i: (0, 0)),
        pl.BlockSpec((D, FH), lambda i: (0, 0)),
        pl.BlockSpec((D, FH), lambda i: (0, 0)),
        pl.BlockSpec((D, FH), lambda i: (0, 0)),
    ],
    out_specs=[
        pl.BlockSpec((NC, NODE_BLK, FH), lambda i: (0, i, 0)),
        pl.BlockSpec((NC, NODE_BLK, FH), lambda i: (0, i, 0)),
    ],
    out_shape=[
        jax.ShapeDtypeStruct((NC, N, FH), jnp.float32),
        jax.ShapeDtypeStruct((NC, N, FH), jnp.float32),
    ],
)

_C_BLK = 8000


def _c_body(ea_ref, wc0_ref, wc1_ref, eb1a_ref, eb1b_ref, c_ref):
    ea = ea_ref[...]
    c_ref[0] = (
        jnp.dot(ea, wc0_ref[...], preferred_element_type=jnp.float32)
        + eb1a_ref[...]
    )
    c_ref[1] = (
        jnp.dot(ea, wc1_ref[...], preferred_element_type=jnp.float32)
        + eb1b_ref[...]
    )


_c_call = pl.pallas_call(
    _c_body,
    grid=(E // _C_BLK,),
    in_specs=[
        pl.BlockSpec((_C_BLK, 5), lambda i: (i, 0)),
        pl.BlockSpec((5, FH), lambda i: (0, 0)),
        pl.BlockSpec((5, FH), lambda i: (0, 0)),
        pl.BlockSpec((1, FH), lambda i: (0, 0)),
        pl.BlockSpec((1, FH), lambda i: (0, 0)),
    ],
    out_specs=pl.BlockSpec((NC, _C_BLK, FH), lambda i: (0, i, 0)),
    out_shape=jax.ShapeDtypeStruct((NC, E, FH), jnp.float32),
)


def _node_body(x_ref, h0_ref, h1_ref, dg_ref, ew2_ref, eb2_ref, nw1x_ref,
               nw1a_ref, nb1_ref, nw2_ref, nb2_ref, out_ref):
    hs = jnp.concatenate([h0_ref[0], h1_ref[0]], axis=1)
    deg = dg_ref[...][:, 0:1]
    agg = (
        jnp.dot(hs, ew2_ref[...], preferred_element_type=jnp.float32)
        + deg * eb2_ref[...]
    )
    z = (
        jnp.dot(x_ref[...], nw1x_ref[...], preferred_element_type=jnp.float32)
        + jnp.dot(agg, nw1a_ref[...], preferred_element_type=jnp.float32)
        + nb1_ref[...]
    )
    h2 = jnp.maximum(z, 0.0)
    out_ref[...] = (
        jnp.dot(h2, nw2_ref[...], preferred_element_type=jnp.float32)
        + nb2_ref[...]
    )


_node_call = pl.pallas_call(
    _node_body,
    grid=(N // NODE_BLK,),
    in_specs=[
        pl.BlockSpec((NODE_BLK, D), lambda i: (i, 0)),
        pl.BlockSpec((1, NODE_BLK, FH), lambda i: (0, i, 0)),
        pl.BlockSpec((1, NODE_BLK, FH), lambda i: (1, i, 0)),
        pl.BlockSpec((NODE_BLK, L), lambda i: (i, 0)),
        pl.BlockSpec((F, F), lambda i: (0, 0)),
        pl.BlockSpec((1, F), lambda i: (0, 0)),
        pl.BlockSpec((D, F), lambda i: (0, 0)),
        pl.BlockSpec((F, F), lambda i: (0, 0)),
        pl.BlockSpec((1, F), lambda i: (0, 0)),
        pl.BlockSpec((F, F), lambda i: (0, 0)),
        pl.BlockSpec((1, F), lambda i: (0, 0)),
    ],
    out_specs=pl.BlockSpec((NODE_BLK, F), lambda i: (i, 0)),
    out_shape=jax.ShapeDtypeStruct((N, F), jnp.float32),
)


def _coordred_body(cr0_ref, cr1_ref, cc0_ref, cc1_ref, pos_ref, co_ref):
    cw = (cr0_ref[0][:, 0:1] + cr1_ref[0][:, 0:1]
          - cc0_ref[0][:, 0:1] - cc1_ref[0][:, 0:1])
    co_ref[...] = jnp.sum(cw * pos_ref[...], axis=0, keepdims=True)[None]


_coordred_call = pl.pallas_call(
    _coordred_body,
    grid=(N // NODE_BLK,),
    in_specs=[
        pl.BlockSpec((1, NODE_BLK, L), lambda i: (0, i, 0)),
        pl.BlockSpec((1, NODE_BLK, L), lambda i: (1, i, 0)),
        pl.BlockSpec((1, NODE_BLK, L), lambda i: (0, i, 0)),
        pl.BlockSpec((1, NODE_BLK, L), lambda i: (1, i, 0)),
        pl.BlockSpec((NODE_BLK, 3), lambda i: (i, 0)),
    ],
    out_specs=pl.BlockSpec((1, 1, 3), lambda i: (i, 0, 0)),
    out_shape=jax.ShapeDtypeStruct((N // NODE_BLK, 1, 3), jnp.float32),
)


_mesh = plsc.VectorSubcoreMesh(
    core_axis_name="c", subcore_axis_name="s", num_cores=NC, num_subcores=NS)

_sc_params = pltpu.CompilerParams(
    needs_layout_passes=False, use_tc_tiling_on_sc=False)


@functools.partial(
    pl.kernel,
    out_type=[
        jax.ShapeDtypeStruct((NC, N, FH), jnp.float32),
        jax.ShapeDtypeStruct((N, L), jnp.float32),
        jax.ShapeDtypeStruct((NC * E,), jnp.float32),
    ],
    mesh=_mesh,
    compiler_params=_sc_params,
    scratch_types=[
        pltpu.VMEM((2, EB), jnp.int32),
        pltpu.VMEM((2, EB), jnp.int32),
        pltpu.VMEM((2, EB), jnp.int32),
        pltpu.VMEM((2, EB), jnp.int32),
        pltpu.VMEM((2, EB), jnp.int32),
        pltpu.VMEM((2, EB, FH), jnp.float32),
        pltpu.VMEM((2, EB, FH), jnp.float32),
        pltpu.VMEM((2, EB, FH), jnp.float32),
        pltpu.VMEM((2, EB, FH), jnp.float32),
        pltpu.VMEM((2, EB), jnp.float32),
        pltpu.VMEM((EB, L), jnp.float32),
        pltpu.VMEM((EB, L), jnp.float32),
        pltpu.VMEM((FH,), jnp.float32),
        pltpu.VMEM_SHARED((NPAD, FH), jnp.float32),
        pltpu.VMEM_SHARED((NPAD, L), jnp.float32),
        pltpu.SemaphoreType.DMA,
        pltpu.SemaphoreType.DMA,
        pltpu.SemaphoreType.DMA,
        pltpu.SemaphoreType.DMA,
        pltpu.SemaphoreType.DMA,
        pltpu.SemaphoreType.DMA,
        pltpu.SemaphoreType.DMA,
        pltpu.SemaphoreType.DMA,
        pltpu.SemaphoreType.DMA,
        pltpu.SemaphoreType.DMA,
        pltpu.SemaphoreType.DMA,
        pltpu.SemaphoreType.DMA,
    ],
)
def _edge_call(rowi_hbm, coli_hbm, a_hbm, b_hbm, c_hbm, w_hbm,
               hs_out, dg_out, s_out,
               rowb_v, colb_v, row2b_v, col2b_v, rscb_v, arb_v, bcb_v, cb_v,
               hb_v, sb_v, ones_v, sacc_v, w_v, hsh, dsh,
               semga0, semga1, semgb0, semgb1, semgc0, semgc1,
               semh0, semh1, semd0, semd1, sems0, sems1):
    cid = lax.axis_index("c")
    sid = lax.axis_index("s")

    pltpu.sync_copy(w_hbm.at[cid], w_v)

    zero16 = jnp.zeros((L,), jnp.float32)
    idx16 = lax.iota(jnp.int32, L)
    onevec = jnp.where(idx16 == 0, 1.0, 0.0).astype(jnp.float32)
    semga = [semga0, semga1]
    semgb = [semgb0, semgb1]
    semgc = [semgc0, semgc1]
    semh = [semh0, semh1]
    semd = [semd0, semd1]
    sems = [sems0, sems1]

    def _zrow(j, carry):
        for kk in range(FH // L):
            hb_v[0, j, pl.ds(kk * L, L)] = zero16
        ones_v[j, :] = zero16
        return carry

    lax.fori_loop(0, EB, _zrow, 0)

    def _zcp(t, carry):
        r0 = sid * RPT + t * EB
        pltpu.sync_copy(hb_v.at[0], hsh.at[pl.ds(r0, EB)])
        pltpu.sync_copy(ones_v, dsh.at[pl.ds(r0, EB)])
        return carry

    lax.fori_loop(0, RPT // EB, _zcp, 0)

    def _orow(j, carry):
        ones_v[j, :] = onevec
        return carry

    lax.fori_loop(0, EB, _orow, 0)

    plsc.subcore_barrier()

    wregs = [w_v[pl.ds(kk * L, L)] for kk in range(FH // L)]
    roff = cid * N
    ebase = sid * EPT

    def _issue(p, bi):
        base = ebase + bi * EB
        pltpu.sync_copy(rowi_hbm.at[pl.ds(base, EB)], rowb_v.at[p])
        pltpu.sync_copy(coli_hbm.at[pl.ds(base, EB)], colb_v.at[p])
        for g in range(EB // L):
            s = pl.ds(g * L, L)
            row2b_v[p, s] = rowb_v[p, s] + roff
            col2b_v[p, s] = colb_v[p, s] + roff
        pltpu.async_copy(a_hbm.at[row2b_v.at[p]], arb_v.at[p], semga[p])
        pltpu.async_copy(b_hbm.at[col2b_v.at[p]], bcb_v.at[p], semgb[p])
        pltpu.async_copy(c_hbm.at[pl.ds(cid * E + base, EB)], cb_v.at[p],
                         semgc[p])

    def _waitg(p, bi):
        base = ebase + bi * EB
        pltpu.make_async_copy(
            a_hbm.at[row2b_v.at[p]], arb_v.at[p], semga[p]).wait()
        pltpu.make_async_copy(
            b_hbm.at[col2b_v.at[p]], bcb_v.at[p], semgb[p]).wait()
        pltpu.make_async_copy(
            c_hbm.at[pl.ds(cid * E + base, EB)], cb_v.at[p], semgc[p]).wait()

    def _compute(p, bi):
        def _row(j, rc):
            acc = zero16
            for kk in range(FH // L):
                s = pl.ds(kk * L, L)
                hv = jnp.maximum(
                    arb_v[p, j, s] + bcb_v[p, j, s] + cb_v[p, j, s], 0.0)
                hb_v[p, j, s] = hv
                acc = acc + hv * wregs[kk]
            sacc_v[j, :] = acc
            return rc

        lax.fori_loop(0, EB, _row, 0)

        for g in range(EB // L):
            eidx = idx16 + g * L
            sv = zero16
            for l in range(L):
                sv = sv + plsc.load_gather(
                    sacc_v, [eidx, jnp.full((L,), l, jnp.int32)])
            sb_v[p, pl.ds(g * L, L)] = sv
            s = pl.ds(g * L, L)
            rscb_v[p, s] = rowb_v[p, s]

    def _scatter(p, bi):
        base = ebase + bi * EB
        pltpu.async_copy(hb_v.at[p], hsh.at[rscb_v.at[p]], semh[p],
                         add=True)

        @pl.when(cid == 0)
        def _deg():
            pltpu.async_copy(ones_v, dsh.at[rscb_v.at[p]], semd[p],
                             add=True)

        pltpu.async_copy(sb_v.at[p], s_out.at[pl.ds(cid * E + base, EB)],
                         sems[p])

    def _drain_sc(p, bi_prev):
        base = ebase + bi_prev * EB
        pltpu.make_async_copy(
            hb_v.at[p], hsh.at[rscb_v.at[p]], semh[p]).wait()

        @pl.when(cid == 0)
        def _deg():
            pltpu.make_async_copy(
                ones_v, dsh.at[rscb_v.at[p]], semd[p]).wait()

        pltpu.make_async_copy(
            sb_v.at[p], s_out.at[pl.ds(cid * E + base, EB)], sems[p]).wait()

    NHALF = NB1 // 2
    _issue(0, 0)
    _issue(1, 1)
    _waitg(0, 0)
    _compute(0, 0)
    _scatter(0, 0)
    _issue(0, 2)
    _waitg(1, 1)
    _compute(1, 1)
    _scatter(1, 1)
    _issue(1, 3)

    def _pair(k, carry):
        for p in range(2):
            bi = 2 * k + p
            _drain_sc(p, bi - 2)
            _waitg(p, bi)
            _compute(p, bi)
            _scatter(p, bi)

            @pl.when(k < NHALF - 1)
            def _next():
                _issue(p, bi + 2)

        return carry

    lax.fori_loop(1, NHALF, _pair, 0)
    _drain_sc(0, NB1 - 2)
    _drain_sc(1, NB1 - 1)

    plsc.subcore_barrier()

    r0 = sid * RPT

    @pl.when(sid < NS - 1)
    def _drain_full():
        pltpu.sync_copy(hsh.at[pl.ds(r0, RPT)], hs_out.at[cid, pl.ds(r0, RPT)])

        @pl.when(cid == 0)
        def _():
            pltpu.sync_copy(dsh.at[pl.ds(r0, RPT)], dg_out.at[pl.ds(r0, RPT)])

    @pl.when(sid == NS - 1)
    def _drain_last():
        pltpu.sync_copy(hsh.at[pl.ds(r0, LAST_ROWS)],
                        hs_out.at[cid, pl.ds(r0, LAST_ROWS)])

        @pl.when(cid == 0)
        def _():
            pltpu.sync_copy(dsh.at[pl.ds(r0, LAST_ROWS)],
                            dg_out.at[pl.ds(r0, LAST_ROWS)])


@functools.partial(
    pl.kernel,
    out_type=[
        jax.ShapeDtypeStruct((NC, N, L), jnp.float32),
        jax.ShapeDtypeStruct((NC, N, L), jnp.float32),
    ],
    mesh=_mesh,
    compiler_params=_sc_params,
    scratch_types=[
        pltpu.VMEM((2, EB), jnp.int32),
        pltpu.VMEM((2, EB), jnp.int32),
        pltpu.VMEM((2, EB), jnp.int32),
        pltpu.VMEM((2, EB), jnp.int32),
        pltpu.VMEM((2, EB), jnp.float32),
        pltpu.VMEM((2, EB), jnp.float32),
        pltpu.VMEM((2, EB, L), jnp.float32),
        pltpu.VMEM((L,), jnp.float32),
        pltpu.VMEM_SHARED((NPAD, L), jnp.float32),
        pltpu.VMEM_SHARED((NPAD, L), jnp.float32),
        pltpu.SemaphoreType.DMA,
        pltpu.SemaphoreType.DMA,
        pltpu.SemaphoreType.DMA,
        pltpu.SemaphoreType.DMA,
        pltpu.SemaphoreType.DMA,
        pltpu.SemaphoreType.DMA,
        pltpu.SemaphoreType.DMA,
        pltpu.SemaphoreType.DMA,
    ],
)
def _coord_call(rowi_hbm, coli_hbm, s_hbm, k_hbm, cr_out, cc_out,
                rowb_v, colb_v, rscb_v, cscb_v, s0b_v, s1b_v, cbufb_v, k_v,
                crsh, ccsh,
                semi0, semi1, sems0, sems1, semr0, semr1, semc0, semc1):
    cid = lax.axis_index("c")
    sid = lax.axis_index("s")
    wid = sid * NC + cid

    pltpu.sync_copy(k_hbm, k_v)
    zero16 = jnp.zeros((L,), jnp.float32)
    idx16 = lax.iota(jnp.int32, L)
    zeroi = jnp.zeros((L,), jnp.int32)
    semi = [semi0, semi1]
    semsv = [sems0, sems1]
    semr = [semr0, semr1]
    semc = [semc0, semc1]

    def _zrow(j, carry):
        for p in range(2):
            cbufb_v[p, j, :] = zero16
        return carry

    lax.fori_loop(0, EB, _zrow, 0)

    def _zcp(t, carry):
        r0 = sid * RPT + t * EB
        pltpu.sync_copy(cbufb_v.at[0], crsh.at[pl.ds(r0, EB)])
        pltpu.sync_copy(cbufb_v.at[0], ccsh.at[pl.ds(r0, EB)])
        return carry

    lax.fori_loop(0, RPT // EB, _zcp, 0)

    plsc.subcore_barrier()

    kvec = k_v[:]
    b0 = kvec[0]
    cw2s = kvec[1]
    cb2s = kvec[2]
    ebase = wid * EPW

    def _issue(p, bi):
        base = ebase + bi * EB
        pltpu.async_copy(rowi_hbm.at[pl.ds(base, EB)], rowb_v.at[p], semi[p])
        pltpu.async_copy(coli_hbm.at[pl.ds(base, EB)], colb_v.at[p], semi[p])
        pltpu.async_copy(s_hbm.at[pl.ds(base, EB)], s0b_v.at[p], semsv[p])
        pltpu.async_copy(s_hbm.at[pl.ds(E + base, EB)], s1b_v.at[p], semsv[p])

    def _waitin(p, bi):
        base = ebase + bi * EB
        pltpu.make_async_copy(
            rowi_hbm.at[pl.ds(base, EB)], rowb_v.at[p], semi[p]).wait()
        pltpu.make_async_copy(
            coli_hbm.at[pl.ds(base, EB)], colb_v.at[p], semi[p]).wait()
        pltpu.make_async_copy(
            s_hbm.at[pl.ds(base, EB)], s0b_v.at[p], semsv[p]).wait()
        pltpu.make_async_copy(
            s_hbm.at[pl.ds(E + base, EB)], s1b_v.at[p], semsv[p]).wait()

    def _compute(p):
        for g in range(EB // L):
            s = pl.ds(g * L, L)
            sv = s0b_v[p, s] + s1b_v[p, s]
            cvec = jnp.maximum(sv + b0, 0.0) * cw2s + cb2s
            plsc.store_scatter(cbufb_v.at[p], [idx16 + g * L, zeroi], cvec)
            rscb_v[p, s] = rowb_v[p, s]
            cscb_v[p, s] = colb_v[p, s]

    def _scatter(p):
        pltpu.async_copy(cbufb_v.at[p], crsh.at[rscb_v.at[p]], semr[p],
                         add=True)
        pltpu.async_copy(cbufb_v.at[p], ccsh.at[cscb_v.at[p]], semc[p],
                         add=True)

    def _drain(p):
        pltpu.make_async_copy(
            cbufb_v.at[p], crsh.at[rscb_v.at[p]], semr[p]).wait()
        pltpu.make_async_copy(
            cbufb_v.at[p], ccsh.at[cscb_v.at[p]], semc[p]).wait()

    NHALF2 = NB2 // 2
    _issue(0, 0)
    _issue(1, 1)
    _waitin(0, 0)
    _compute(0)
    _scatter(0)
    _issue(0, 2)
    _waitin(1, 1)
    _compute(1)
    _scatter(1)
    _issue(1, 3)

    def _pair(k, carry):
        for p in range(2):
            bi = 2 * k + p
            _drain(p)
            _waitin(p, bi)
            _compute(p)
            _scatter(p)

            @pl.when(bi + 2 < NB2)
            def _next():
                _issue(p, bi + 2)

        return carry

    lax.fori_loop(1, NHALF2, _pair, 0)
    if NB2 % 2 == 1:
        _drain(0)
        _waitin(0, NB2 - 1)
        _compute(0)
        _scatter(0)
    _drain(0)
    _drain(1)

    plsc.subcore_barrier()

    r0 = sid * RPT

    @pl.when(sid < NS - 1)
    def _drain_full():
        pltpu.sync_copy(crsh.at[pl.ds(r0, RPT)], cr_out.at[cid, pl.ds(r0, RPT)])
        pltpu.sync_copy(ccsh.at[pl.ds(r0, RPT)], cc_out.at[cid, pl.ds(r0, RPT)])

    @pl.when(sid == NS - 1)
    def _drain_last():
        pltpu.sync_copy(crsh.at[pl.ds(r0, LAST_ROWS)],
                        cr_out.at[cid, pl.ds(r0, LAST_ROWS)])
        pltpu.sync_copy(ccsh.at[pl.ds(r0, LAST_ROWS)],
                        cc_out.at[cid, pl.ds(r0, LAST_ROWS)])


def kernel(x, edge_index, edge_attr, pos,
           ew1, eb1, ew2, eb2,
           nw1, nb1, nw2, nb2,
           cw1, cb1, cw2, cb2):
    rowi = edge_index[0].astype(jnp.int32)
    coli = edge_index[1].astype(jnp.int32)

    a_st, b_st = _ab_call(
        x, ew1[:D, :FH], ew1[:D, FH:], ew1[D:2 * D, :FH], ew1[D:2 * D, FH:])
    c_st = _c_call(
        edge_attr, ew1[2 * D:, :FH], ew1[2 * D:, FH:],
        eb1[:FH].reshape(1, FH), eb1[FH:].reshape(1, FH))

    w = (ew2 @ cw1)[:, 0]
    b0 = eb2 @ cw1[:, 0] + cb1[0]
    consts = jnp.concatenate([
        jnp.reshape(b0, (1,)), jnp.reshape(cw2, (1,)), jnp.reshape(cb2, (1,)),
        jnp.zeros((L - 3,), jnp.float32),
    ])

    hs, dg, s_parts = _edge_call(
        rowi, coli,
        a_st.reshape(NC * N, FH), b_st.reshape(NC * N, FH),
        c_st.reshape(NC * E, FH), w.reshape(NC, FH))

    crow, ccol = _coord_call(rowi, coli, s_parts, consts)

    node_features = _node_call(
        x, hs, hs, dg, ew2, eb2.reshape(1, F),
        nw1[:D], nw1[D:], nb1.reshape(1, F), nw2, nb2.reshape(1, F))
    co_parts = _coordred_call(crow, crow, ccol, ccol, pos)

    coord = co_parts.reshape(N // NODE_BLK, 3).sum(axis=0)
    pos_out = pos + coord[None, :]
    return node_features, pos_out


# speedup vs baseline: 1.1028x; 1.0011x over previous
import functools

import jax
import jax.numpy as jnp
import numpy as np
from jax import lax
from jax.experimental import pallas as pl
from jax.experimental.pallas import tpu as pltpu
from jax.experimental.pallas import tpu_sc as plsc

N = 10000
E = 320000
D = 128
F = 128

NC = 2
NS = 16
L = 16
FH = F // NC
NCH = F // FH

EB = 80
EPT = E // NS
NB1 = EPT // EB
NW = NC * NS
EPW = E // NW
NB2 = EPW // EB

NPAD = 10240
RPT = NPAD // NS
LAST_ROWS = N - (NS - 1) * RPT
NODE_BLK = 1000

_PCORE = np.concatenate([
    np.concatenate([np.arange(0, 32, 2), np.arange(1, 32, 2)]) + 32 * b
    for b in range(FH // 32)
])
_PFULL = np.concatenate([_PCORE, _PCORE + FH])


def _ab_body(x_ref, wa0_ref, wa1_ref, wb0_ref, wb1_ref, a_ref, b_ref):
    xb = x_ref[...]
    a_ref[0] = jnp.dot(xb, wa0_ref[...], preferred_element_type=jnp.float32)
    a_ref[1] = jnp.dot(xb, wa1_ref[...], preferred_element_type=jnp.float32)
    b_ref[0] = jnp.dot(xb, wb0_ref[...], preferred_element_type=jnp.float32)
    b_ref[1] = jnp.dot(xb, wb1_ref[...], preferred_element_type=jnp.float32)


_ab_call = pl.pallas_call(
    _ab_body,
    grid=(N // NODE_BLK,),
    in_specs=[
        pl.BlockSpec((NODE_BLK, D), lambda i: (i, 0)),
        pl.BlockSpec((D, FH), lambda i: (0, 0)),
        pl.BlockSpec((D, FH), lambda i: (0, 0)),
        pl.BlockSpec((D, FH), lambda i: (0, 0)),
        pl.BlockSpec((D, FH), lambda i: (0, 0)),
    ],
    out_specs=[
        pl.BlockSpec((NC, NODE_BLK, FH), lambda i: (0, i, 0)),
        pl.BlockSpec((NC, NODE_BLK, FH), lambda i: (0, i, 0)),
    ],
    out_shape=[
        jax.ShapeDtypeStruct((NC, N, FH), jnp.float32),
        jax.ShapeDtypeStruct((NC, N, FH), jnp.float32),
    ],
)

_C_BLK = 8000


def _c_body(ea_ref, wc0_ref, wc1_ref, eb1a_ref, eb1b_ref, c_ref):
    ea = ea_ref[...]
    c_ref[0] = (
        jnp.dot(ea, wc0_ref[...], preferred_element_type=jnp.float32)
        + eb1a_ref[...]
    )
    c_ref[1] = (
        jnp.dot(ea, wc1_ref[...], preferred_element_type=jnp.float32)
        + eb1b_ref[...]
    )


_c_call = pl.pallas_call(
    _c_body,
    grid=(E // _C_BLK,),
    in_specs=[
        pl.BlockSpec((_C_BLK, 5), lambda i: (i, 0)),
        pl.BlockSpec((5, FH), lambda i: (0, 0)),
        pl.BlockSpec((5, FH), lambda i: (0, 0)),
        pl.BlockSpec((1, FH), lambda i: (0, 0)),
        pl.BlockSpec((1, FH), lambda i: (0, 0)),
    ],
    out_specs=pl.BlockSpec((NC, _C_BLK, FH), lambda i: (0, i, 0)),
    out_shape=jax.ShapeDtypeStruct((NC, E, FH), jnp.float32),
)


def _node_body(x_ref, h0_ref, h1_ref, dg_ref, ew2_ref, eb2_ref, nw1x_ref,
               nw1a_ref, nb1_ref, nw2_ref, nb2_ref, out_ref):
    hs = jnp.concatenate([h0_ref[0], h1_ref[0]], axis=1)
    deg = dg_ref[...][:, 0:1]
    agg = (
        jnp.dot(hs, ew2_ref[...], preferred_element_type=jnp.float32)
        + deg * eb2_ref[...]
    )
    z = (
        jnp.dot(x_ref[...], nw1x_ref[...], preferred_element_type=jnp.float32)
        + jnp.dot(agg, nw1a_ref[...], preferred_element_type=jnp.float32)
        + nb1_ref[...]
    )
    h2 = jnp.maximum(z, 0.0)
    out_ref[...] = (
        jnp.dot(h2, nw2_ref[...], preferred_element_type=jnp.float32)
        + nb2_ref[...]
    )


_node_call = pl.pallas_call(
    _node_body,
    grid=(N // NODE_BLK,),
    in_specs=[
        pl.BlockSpec((NODE_BLK, D), lambda i: (i, 0)),
        pl.BlockSpec((1, NODE_BLK, FH), lambda i: (0, i, 0)),
        pl.BlockSpec((1, NODE_BLK, FH), lambda i: (1, i, 0)),
        pl.BlockSpec((NODE_BLK, L), lambda i: (i, 0)),
        pl.BlockSpec((F, F), lambda i: (0, 0)),
        pl.BlockSpec((1, F), lambda i: (0, 0)),
        pl.BlockSpec((D, F), lambda i: (0, 0)),
        pl.BlockSpec((F, F), lambda i: (0, 0)),
        pl.BlockSpec((1, F), lambda i: (0, 0)),
        pl.BlockSpec((F, F), lambda i: (0, 0)),
        pl.BlockSpec((1, F), lambda i: (0, 0)),
    ],
    out_specs=pl.BlockSpec((NODE_BLK, F), lambda i: (i, 0)),
    out_shape=jax.ShapeDtypeStruct((N, F), jnp.float32),
)


def _coordred_body(cr0_ref, cr1_ref, cc0_ref, cc1_ref, pos_ref, co_ref):
    cw = (cr0_ref[0][:, 0:1] + cr1_ref[0][:, 0:1]
          - cc0_ref[0][:, 0:1] - cc1_ref[0][:, 0:1])
    co_ref[...] = jnp.sum(cw * pos_ref[...], axis=0, keepdims=True)[None]


_coordred_call = pl.pallas_call(
    _coordred_body,
    grid=(N // NODE_BLK,),
    in_specs=[
        pl.BlockSpec((1, NODE_BLK, L), lambda i: (0, i, 0)),
        pl.BlockSpec((1, NODE_BLK, L), lambda i: (1, i, 0)),
        pl.BlockSpec((1, NODE_BLK, L), lambda i: (0, i, 0)),
        pl.BlockSpec((1, NODE_BLK, L), lambda i: (1, i, 0)),
        pl.BlockSpec((NODE_BLK, 3), lambda i: (i, 0)),
    ],
    out_specs=pl.BlockSpec((1, 1, 3), lambda i: (i, 0, 0)),
    out_shape=jax.ShapeDtypeStruct((N // NODE_BLK, 1, 3), jnp.float32),
)


_mesh = plsc.VectorSubcoreMesh(
    core_axis_name="c", subcore_axis_name="s", num_cores=NC, num_subcores=NS)

_sc_params = pltpu.CompilerParams(
    needs_layout_passes=False, use_tc_tiling_on_sc=False)


@functools.partial(
    pl.kernel,
    out_type=[
        jax.ShapeDtypeStruct((NC, N, FH), jnp.float32),
        jax.ShapeDtypeStruct((N, L), jnp.float32),
        jax.ShapeDtypeStruct((NC * E,), jnp.float32),
    ],
    mesh=_mesh,
    compiler_params=_sc_params,
    scratch_types=[
        pltpu.VMEM((2, EB), jnp.int32),
        pltpu.VMEM((2, EB), jnp.int32),
        pltpu.VMEM((2, EB), jnp.int32),
        pltpu.VMEM((2, EB), jnp.int32),
        pltpu.VMEM((2, EB), jnp.int32),
        pltpu.VMEM((2, EB, FH), jnp.float32),
        pltpu.VMEM((2, EB, FH), jnp.float32),
        pltpu.VMEM((2, EB, FH), jnp.float32),
        pltpu.VMEM((2, EB, FH), jnp.float32),
        pltpu.VMEM((2, EB), jnp.float32),
        pltpu.VMEM((EB, L), jnp.float32),
        pltpu.VMEM((EB, L), jnp.float32),
        pltpu.VMEM((FH,), jnp.float32),
        pltpu.VMEM_SHARED((NPAD, FH), jnp.float32),
        pltpu.VMEM_SHARED((NPAD, L), jnp.float32),
        pltpu.SemaphoreType.DMA,
        pltpu.SemaphoreType.DMA,
        pltpu.SemaphoreType.DMA,
        pltpu.SemaphoreType.DMA,
        pltpu.SemaphoreType.DMA,
        pltpu.SemaphoreType.DMA,
        pltpu.SemaphoreType.DMA,
        pltpu.SemaphoreType.DMA,
        pltpu.SemaphoreType.DMA,
        pltpu.SemaphoreType.DMA,
        pltpu.SemaphoreType.DMA,
        pltpu.SemaphoreType.DMA,
    ],
)
def _edge_call(rowi_hbm, coli_hbm, a_hbm, b_hbm, c_hbm, w_hbm,
               hs_out, dg_out, s_out,
               rowb_v, colb_v, row2b_v, col2b_v, rscb_v, arb_v, bcb_v, cb_v,
               hb_v, sb_v, ones_v, sacc_v, w_v, hsh, dsh,
               semga0, semga1, semgb0, semgb1, semgc0, semgc1,
               semh0, semh1, semd0, semd1, sems0, sems1):
    cid = lax.axis_index("c")
    sid = lax.axis_index("s")

    pltpu.sync_copy(w_hbm.at[cid], w_v)

    zero16 = jnp.zeros((L,), jnp.float32)
    idx16 = lax.iota(jnp.int32, L)
    onevec = jnp.where(idx16 == 0, 1.0, 0.0).astype(jnp.float32)
    semga = [semga0, semga1]
    semgb = [semgb0, semgb1]
    semgc = [semgc0, semgc1]
    semh = [semh0, semh1]
    semd = [semd0, semd1]
    sems = [sems0, sems1]

    def _zrow(j, carry):
        for kk in range(FH // L):
            hb_v[0, j, pl.ds(kk * L, L)] = zero16
        ones_v[j, :] = zero16
        return carry

    lax.fori_loop(0, EB, _zrow, 0)

    def _zcp(t, carry):
        r0 = sid * RPT + t * EB
        pltpu.async_copy(hb_v.at[0], hsh.at[pl.ds(r0, EB)], semga0)
        pltpu.async_copy(ones_v, dsh.at[pl.ds(r0, EB)], semgb0)
        return carry

    lax.fori_loop(0, RPT // EB, _zcp, 0)

    def _zwait(t, carry):
        r0 = sid * RPT + t * EB
        pltpu.make_async_copy(
            hb_v.at[0], hsh.at[pl.ds(r0, EB)], semga0).wait()
        pltpu.make_async_copy(
            ones_v, dsh.at[pl.ds(r0, EB)], semgb0).wait()
        return carry

    lax.fori_loop(0, RPT // EB, _zwait, 0)

    def _orow(j, carry):
        ones_v[j, :] = onevec
        return carry

    lax.fori_loop(0, EB, _orow, 0)

    plsc.subcore_barrier()

    wregs = [w_v[pl.ds(kk * L, L)] for kk in range(FH // L)]
    roff = cid * N
    ebase = sid * EPT

    def _issue(p, bi):
        base = ebase + bi * EB
        pltpu.sync_copy(rowi_hbm.at[pl.ds(base, EB)], rowb_v.at[p])
        pltpu.sync_copy(coli_hbm.at[pl.ds(base, EB)], colb_v.at[p])
        for g in range(EB // L):
            s = pl.ds(g * L, L)
            row2b_v[p, s] = rowb_v[p, s] + roff
            col2b_v[p, s] = colb_v[p, s] + roff
        pltpu.async_copy(a_hbm.at[row2b_v.at[p]], arb_v.at[p], semga[p])
        pltpu.async_copy(b_hbm.at[col2b_v.at[p]], bcb_v.at[p], semgb[p])
        pltpu.async_copy(c_hbm.at[pl.ds(cid * E + base, EB)], cb_v.at[p],
                         semgc[p])

    def _waitg(p, bi):
        base = ebase + bi * EB
        pltpu.make_async_copy(
            a_hbm.at[row2b_v.at[p]], arb_v.at[p], semga[p]).wait()
        pltpu.make_async_copy(
            b_hbm.at[col2b_v.at[p]], bcb_v.at[p], semgb[p]).wait()
        pltpu.make_async_copy(
            c_hbm.at[pl.ds(cid * E + base, EB)], cb_v.at[p], semgc[p]).wait()

    def _compute(p, bi):
        def _row(j, rc):
            acc = zero16
            for kk in range(FH // L):
                s = pl.ds(kk * L, L)
                hv = jnp.maximum(
                    arb_v[p, j, s] + bcb_v[p, j, s] + cb_v[p, j, s], 0.0)
                hb_v[p, j, s] = hv
                acc = acc + hv * wregs[kk]
            sacc_v[j, :] = acc
            return rc

        lax.fori_loop(0, EB, _row, 0)

        for g in range(EB // L):
            eidx = idx16 + g * L
            sv = zero16
            for l in range(L):
                sv = sv + plsc.load_gather(
                    sacc_v, [eidx, jnp.full((L,), l, jnp.int32)])
            sb_v[p, pl.ds(g * L, L)] = sv
            s = pl.ds(g * L, L)
            rscb_v[p, s] = rowb_v[p, s]

    def _scatter(p, bi):
        base = ebase + bi * EB
        pltpu.async_copy(hb_v.at[p], hsh.at[rscb_v.at[p]], semh[p],
                         add=True)

        @pl.when(cid == 0)
        def _deg():
            pltpu.async_copy(ones_v, dsh.at[rscb_v.at[p]], semd[p],
                             add=True)

        pltpu.async_copy(sb_v.at[p], s_out.at[pl.ds(cid * E + base, EB)],
                         sems[p])

    def _drain_sc(p, bi_prev):
        base = ebase + bi_prev * EB
        pltpu.make_async_copy(
            hb_v.at[p], hsh.at[rscb_v.at[p]], semh[p]).wait()

        @pl.when(cid == 0)
        def _deg():
            pltpu.make_async_copy(
                ones_v, dsh.at[rscb_v.at[p]], semd[p]).wait()

        pltpu.make_async_copy(
            sb_v.at[p], s_out.at[pl.ds(cid * E + base, EB)], sems[p]).wait()

    NHALF = NB1 // 2
    _issue(0, 0)
    _issue(1, 1)
    _waitg(0, 0)
    _compute(0, 0)
    _scatter(0, 0)
    _issue(0, 2)
    _waitg(1, 1)
    _compute(1, 1)
    _scatter(1, 1)
    _issue(1, 3)

    def _pair(k, carry):
        for p in range(2):
            bi = 2 * k + p
            _drain_sc(p, bi - 2)
            _waitg(p, bi)
            _compute(p, bi)
            _scatter(p, bi)

            @pl.when(k < NHALF - 1)
            def _next():
                _issue(p, bi + 2)

        return carry

    lax.fori_loop(1, NHALF, _pair, 0)
    _drain_sc(0, NB1 - 2)
    _drain_sc(1, NB1 - 1)

    plsc.subcore_barrier()

    r0 = sid * RPT

    @pl.when(sid < NS - 1)
    def _drain_full():
        pltpu.sync_copy(hsh.at[pl.ds(r0, RPT)], hs_out.at[cid, pl.ds(r0, RPT)])

        @pl.when(cid == 0)
        def _():
            pltpu.sync_copy(dsh.at[pl.ds(r0, RPT)], dg_out.at[pl.ds(r0, RPT)])

    @pl.when(sid == NS - 1)
    def _drain_last():
        pltpu.sync_copy(hsh.at[pl.ds(r0, LAST_ROWS)],
                        hs_out.at[cid, pl.ds(r0, LAST_ROWS)])

        @pl.when(cid == 0)
        def _():
            pltpu.sync_copy(dsh.at[pl.ds(r0, LAST_ROWS)],
                            dg_out.at[pl.ds(r0, LAST_ROWS)])


@functools.partial(
    pl.kernel,
    out_type=[
        jax.ShapeDtypeStruct((NC, N, L), jnp.float32),
        jax.ShapeDtypeStruct((NC, N, L), jnp.float32),
    ],
    mesh=_mesh,
    compiler_params=_sc_params,
    scratch_types=[
        pltpu.VMEM((2, EB), jnp.int32),
        pltpu.VMEM((2, EB), jnp.int32),
        pltpu.VMEM((2, EB), jnp.int32),
        pltpu.VMEM((2, EB), jnp.int32),
        pltpu.VMEM((2, EB), jnp.float32),
        pltpu.VMEM((2, EB), jnp.float32),
        pltpu.VMEM((2, EB, L), jnp.float32),
        pltpu.VMEM((L,), jnp.float32),
        pltpu.VMEM_SHARED((NPAD, L), jnp.float32),
        pltpu.VMEM_SHARED((NPAD, L), jnp.float32),
        pltpu.SemaphoreType.DMA,
        pltpu.SemaphoreType.DMA,
        pltpu.SemaphoreType.DMA,
        pltpu.SemaphoreType.DMA,
        pltpu.SemaphoreType.DMA,
        pltpu.SemaphoreType.DMA,
        pltpu.SemaphoreType.DMA,
        pltpu.SemaphoreType.DMA,
    ],
)
def _coord_call(rowi_hbm, coli_hbm, s_hbm, k_hbm, cr_out, cc_out,
                rowb_v, colb_v, rscb_v, cscb_v, s0b_v, s1b_v, cbufb_v, k_v,
                crsh, ccsh,
                semi0, semi1, sems0, sems1, semr0, semr1, semc0, semc1):
    cid = lax.axis_index("c")
    sid = lax.axis_index("s")
    wid = sid * NC + cid

    pltpu.sync_copy(k_hbm, k_v)
    zero16 = jnp.zeros((L,), jnp.float32)
    idx16 = lax.iota(jnp.int32, L)
    zeroi = jnp.zeros((L,), jnp.int32)
    semi = [semi0, semi1]
    semsv = [sems0, sems1]
    semr = [semr0, semr1]
    semc = [semc0, semc1]

    def _zrow(j, carry):
        for p in range(2):
            cbufb_v[p, j, :] = zero16
        return carry

    lax.fori_loop(0, EB, _zrow, 0)

    def _zcp(t, carry):
        r0 = sid * RPT + t * EB
        pltpu.async_copy(cbufb_v.at[0], crsh.at[pl.ds(r0, EB)], semi0)
        pltpu.async_copy(cbufb_v.at[0], ccsh.at[pl.ds(r0, EB)], sems0)
        return carry

    lax.fori_loop(0, RPT // EB, _zcp, 0)

    def _zwait(t, carry):
        r0 = sid * RPT + t * EB
        pltpu.make_async_copy(
            cbufb_v.at[0], crsh.at[pl.ds(r0, EB)], semi0).wait()
        pltpu.make_async_copy(
            cbufb_v.at[0], ccsh.at[pl.ds(r0, EB)], sems0).wait()
        return carry

    lax.fori_loop(0, RPT // EB, _zwait, 0)

    plsc.subcore_barrier()

    kvec = k_v[:]
    b0 = kvec[0]
    cw2s = kvec[1]
    cb2s = kvec[2]
    ebase = wid * EPW

    def _issue(p, bi):
        base = ebase + bi * EB
        pltpu.async_copy(rowi_hbm.at[pl.ds(base, EB)], rowb_v.at[p], semi[p])
        pltpu.async_copy(coli_hbm.at[pl.ds(base, EB)], colb_v.at[p], semi[p])
        pltpu.async_copy(s_hbm.at[pl.ds(base, EB)], s0b_v.at[p], semsv[p])
        pltpu.async_copy(s_hbm.at[pl.ds(E + base, EB)], s1b_v.at[p], semsv[p])

    def _waitin(p, bi):
        base = ebase + bi * EB
        pltpu.make_async_copy(
            rowi_hbm.at[pl.ds(base, EB)], rowb_v.at[p], semi[p]).wait()
        pltpu.make_async_copy(
            coli_hbm.at[pl.ds(base, EB)], colb_v.at[p], semi[p]).wait()
        pltpu.make_async_copy(
            s_hbm.at[pl.ds(base, EB)], s0b_v.at[p], semsv[p]).wait()
        pltpu.make_async_copy(
            s_hbm.at[pl.ds(E + base, EB)], s1b_v.at[p], semsv[p]).wait()

    def _compute(p):
        for g in range(EB // L):
            s = pl.ds(g * L, L)
            sv = s0b_v[p, s] + s1b_v[p, s]
            cvec = jnp.maximum(sv + b0, 0.0) * cw2s + cb2s
            plsc.store_scatter(cbufb_v.at[p], [idx16 + g * L, zeroi], cvec)
            rscb_v[p, s] = rowb_v[p, s]
            cscb_v[p, s] = colb_v[p, s]

    def _scatter(p):
        pltpu.async_copy(cbufb_v.at[p], crsh.at[rscb_v.at[p]], semr[p],
                         add=True)
        pltpu.async_copy(cbufb_v.at[p], ccsh.at[cscb_v.at[p]], semc[p],
                         add=True)

    def _drain(p):
        pltpu.make_async_copy(
            cbufb_v.at[p], crsh.at[rscb_v.at[p]], semr[p]).wait()
        pltpu.make_async_copy(
            cbufb_v.at[p], ccsh.at[cscb_v.at[p]], semc[p]).wait()

    NHALF2 = NB2 // 2
    _issue(0, 0)
    _issue(1, 1)
    _waitin(0, 0)
    _compute(0)
    _scatter(0)
    _issue(0, 2)
    _waitin(1, 1)
    _compute(1)
    _scatter(1)
    _issue(1, 3)

    def _pair(k, carry):
        for p in range(2):
            bi = 2 * k + p
            _drain(p)
            _waitin(p, bi)
            _compute(p)
            _scatter(p)

            @pl.when(bi + 2 < NB2)
            def _next():
                _issue(p, bi + 2)

        return carry

    lax.fori_loop(1, NHALF2, _pair, 0)
    if NB2 % 2 == 1:
        _drain(0)
        _waitin(0, NB2 - 1)
        _compute(0)
        _scatter(0)
    _drain(0)
    _drain(1)

    plsc.subcore_barrier()

    r0 = sid * RPT

    @pl.when(sid < NS - 1)
    def _drain_full():
        pltpu.sync_copy(crsh.at[pl.ds(r0, RPT)], cr_out.at[cid, pl.ds(r0, RPT)])
        pltpu.sync_copy(ccsh.at[pl.ds(r0, RPT)], cc_out.at[cid, pl.ds(r0, RPT)])

    @pl.when(sid == NS - 1)
    def _drain_last():
        pltpu.sync_copy(crsh.at[pl.ds(r0, LAST_ROWS)],
                        cr_out.at[cid, pl.ds(r0, LAST_ROWS)])
        pltpu.sync_copy(ccsh.at[pl.ds(r0, LAST_ROWS)],
                        cc_out.at[cid, pl.ds(r0, LAST_ROWS)])


def kernel(x, edge_index, edge_attr, pos,
           ew1, eb1, ew2, eb2,
           nw1, nb1, nw2, nb2,
           cw1, cb1, cw2, cb2):
    rowi = edge_index[0].astype(jnp.int32)
    coli = edge_index[1].astype(jnp.int32)

    a_st, b_st = _ab_call(
        x, ew1[:D, :FH], ew1[:D, FH:], ew1[D:2 * D, :FH], ew1[D:2 * D, FH:])
    c_st = _c_call(
        edge_attr, ew1[2 * D:, :FH], ew1[2 * D:, FH:],
        eb1[:FH].reshape(1, FH), eb1[FH:].reshape(1, FH))

    w = (ew2 @ cw1)[:, 0]
    b0 = eb2 @ cw1[:, 0] + cb1[0]
    consts = jnp.concatenate([
        jnp.reshape(b0, (1,)), jnp.reshape(cw2, (1,)), jnp.reshape(cb2, (1,)),
        jnp.zeros((L - 3,), jnp.float32),
    ])

    hs, dg, s_parts = _edge_call(
        rowi, coli,
        a_st.reshape(NC * N, FH), b_st.reshape(NC * N, FH),
        c_st.reshape(NC * E, FH), w.reshape(NC, FH))

    crow, ccol = _coord_call(rowi, coli, s_parts, consts)

    node_features = _node_call(
        x, hs, hs, dg, ew2, eb2.reshape(1, F),
        nw1[:D], nw1[D:], nb1.reshape(1, F), nw2, nb2.reshape(1, F))
    co_parts = _coordred_call(crow, crow, ccol, ccol, pos)

    coord = co_parts.reshape(N // NODE_BLK, 3).sum(axis=0)
    pos_out = pos + coord[None, :]
    return node_features, pos_out


# speedup vs baseline: 1.1050x; 1.0020x over previous
import functools

import jax
import jax.numpy as jnp
import numpy as np
from jax import lax
from jax.experimental import pallas as pl
from jax.experimental.pallas import tpu as pltpu
from jax.experimental.pallas import tpu_sc as plsc

N = 10000
E = 320000
D = 128
F = 128

NC = 2
NS = 16
L = 16
FH = F // NC
NCH = F // FH

EB = 80
EPT = E // NS
NB1 = EPT // EB
NW = NC * NS
EPW = E // NW
NB2 = EPW // EB

NPAD = 10240
RPT = NPAD // NS
LAST_ROWS = N - (NS - 1) * RPT
NODE_BLK = 1000

_PCORE = np.concatenate([
    np.concatenate([np.arange(0, 32, 2), np.arange(1, 32, 2)]) + 32 * b
    for b in range(FH // 32)
])
_PFULL = np.concatenate([_PCORE, _PCORE + FH])


_C_BLK = 8000
_AB_STEPS = N // NODE_BLK


def _abc_body(x_ref, wa0_ref, wa1_ref, wb0_ref, wb1_ref, ea_ref, wc0_ref,
              wc1_ref, eb1a_ref, eb1b_ref, a_ref, b_ref, c_ref):
    i = pl.program_id(0)

    @pl.when(i < _AB_STEPS)
    def _ab():
        xb = x_ref[...]
        a_ref[0] = jnp.dot(
            xb, wa0_ref[...], preferred_element_type=jnp.float32)
        a_ref[1] = jnp.dot(
            xb, wa1_ref[...], preferred_element_type=jnp.float32)
        b_ref[0] = jnp.dot(
            xb, wb0_ref[...], preferred_element_type=jnp.float32)
        b_ref[1] = jnp.dot(
            xb, wb1_ref[...], preferred_element_type=jnp.float32)

    ea = ea_ref[...]
    c_ref[0] = (
        jnp.dot(ea, wc0_ref[...], preferred_element_type=jnp.float32)
        + eb1a_ref[...]
    )
    c_ref[1] = (
        jnp.dot(ea, wc1_ref[...], preferred_element_type=jnp.float32)
        + eb1b_ref[...]
    )


def _clampi(i):
    return jnp.minimum(i, _AB_STEPS - 1)


_abc_call = pl.pallas_call(
    _abc_body,
    grid=(E // _C_BLK,),
    in_specs=[
        pl.BlockSpec((NODE_BLK, D), lambda i: (_clampi(i), 0)),
        pl.BlockSpec((D, FH), lambda i: (0, 0)),
        pl.BlockSpec((D, FH), lambda i: (0, 0)),
        pl.BlockSpec((D, FH), lambda i: (0, 0)),
        pl.BlockSpec((D, FH), lambda i: (0, 0)),
        pl.BlockSpec((_C_BLK, 5), lambda i: (i, 0)),
        pl.BlockSpec((5, FH), lambda i: (0, 0)),
        pl.BlockSpec((5, FH), lambda i: (0, 0)),
        pl.BlockSpec((1, FH), lambda i: (0, 0)),
        pl.BlockSpec((1, FH), lambda i: (0, 0)),
    ],
    out_specs=[
        pl.BlockSpec((NC, NODE_BLK, FH), lambda i: (0, _clampi(i), 0)),
        pl.BlockSpec((NC, NODE_BLK, FH), lambda i: (0, _clampi(i), 0)),
        pl.BlockSpec((NC, _C_BLK, FH), lambda i: (0, i, 0)),
    ],
    out_shape=[
        jax.ShapeDtypeStruct((NC, N, FH), jnp.float32),
        jax.ShapeDtypeStruct((NC, N, FH), jnp.float32),
        jax.ShapeDtypeStruct((NC, E, FH), jnp.float32),
    ],
)


def _node_body(x_ref, h0_ref, h1_ref, dg_ref, ew2_ref, eb2_ref, nw1x_ref,
               nw1a_ref, nb1_ref, nw2_ref, nb2_ref, out_ref):
    hs = jnp.concatenate([h0_ref[0], h1_ref[0]], axis=1)
    deg = dg_ref[...][:, 0:1]
    agg = (
        jnp.dot(hs, ew2_ref[...], preferred_element_type=jnp.float32)
        + deg * eb2_ref[...]
    )
    z = (
        jnp.dot(x_ref[...], nw1x_ref[...], preferred_element_type=jnp.float32)
        + jnp.dot(agg, nw1a_ref[...], preferred_element_type=jnp.float32)
        + nb1_ref[...]
    )
    h2 = jnp.maximum(z, 0.0)
    out_ref[...] = (
        jnp.dot(h2, nw2_ref[...], preferred_element_type=jnp.float32)
        + nb2_ref[...]
    )


_node_call = pl.pallas_call(
    _node_body,
    grid=(N // NODE_BLK,),
    in_specs=[
        pl.BlockSpec((NODE_BLK, D), lambda i: (i, 0)),
        pl.BlockSpec((1, NODE_BLK, FH), lambda i: (0, i, 0)),
        pl.BlockSpec((1, NODE_BLK, FH), lambda i: (1, i, 0)),
        pl.BlockSpec((NODE_BLK, L), lambda i: (i, 0)),
        pl.BlockSpec((F, F), lambda i: (0, 0)),
        pl.BlockSpec((1, F), lambda i: (0, 0)),
        pl.BlockSpec((D, F), lambda i: (0, 0)),
        pl.BlockSpec((F, F), lambda i: (0, 0)),
        pl.BlockSpec((1, F), lambda i: (0, 0)),
        pl.BlockSpec((F, F), lambda i: (0, 0)),
        pl.BlockSpec((1, F), lambda i: (0, 0)),
    ],
    out_specs=pl.BlockSpec((NODE_BLK, F), lambda i: (i, 0)),
    out_shape=jax.ShapeDtypeStruct((N, F), jnp.float32),
)


def _coordred_body(cr0_ref, cr1_ref, cc0_ref, cc1_ref, pos_ref, co_ref):
    cw = (cr0_ref[0][:, 0:1] + cr1_ref[0][:, 0:1]
          - cc0_ref[0][:, 0:1] - cc1_ref[0][:, 0:1])
    co_ref[...] = jnp.sum(cw * pos_ref[...], axis=0, keepdims=True)[None]


_coordred_call = pl.pallas_call(
    _coordred_body,
    grid=(N // NODE_BLK,),
    in_specs=[
        pl.BlockSpec((1, NODE_BLK, L), lambda i: (0, i, 0)),
        pl.BlockSpec((1, NODE_BLK, L), lambda i: (1, i, 0)),
        pl.BlockSpec((1, NODE_BLK, L), lambda i: (0, i, 0)),
        pl.BlockSpec((1, NODE_BLK, L), lambda i: (1, i, 0)),
        pl.BlockSpec((NODE_BLK, 3), lambda i: (i, 0)),
    ],
    out_specs=pl.BlockSpec((1, 1, 3), lambda i: (i, 0, 0)),
    out_shape=jax.ShapeDtypeStruct((N // NODE_BLK, 1, 3), jnp.float32),
)


_mesh = plsc.VectorSubcoreMesh(
    core_axis_name="c", subcore_axis_name="s", num_cores=NC, num_subcores=NS)

_sc_params = pltpu.CompilerParams(
    needs_layout_passes=False, use_tc_tiling_on_sc=False)


@functools.partial(
    pl.kernel,
    out_type=[
        jax.ShapeDtypeStruct((NC, N, FH), jnp.float32),
        jax.ShapeDtypeStruct((N, L), jnp.float32),
        jax.ShapeDtypeStruct((NC * E,), jnp.float32),
    ],
    mesh=_mesh,
    compiler_params=_sc_params,
    scratch_types=[
        pltpu.VMEM((2, EB), jnp.int32),
        pltpu.VMEM((2, EB), jnp.int32),
        pltpu.VMEM((2, EB), jnp.int32),
        pltpu.VMEM((2, EB), jnp.int32),
        pltpu.VMEM((2, EB), jnp.int32),
        pltpu.VMEM((2, EB, FH), jnp.float32),
        pltpu.VMEM((2, EB, FH), jnp.float32),
        pltpu.VMEM((2, EB, FH), jnp.float32),
        pltpu.VMEM((2, EB, FH), jnp.float32),
        pltpu.VMEM((2, EB), jnp.float32),
        pltpu.VMEM((EB, L), jnp.float32),
        pltpu.VMEM((EB, L), jnp.float32),
        pltpu.VMEM((FH,), jnp.float32),
        pltpu.VMEM_SHARED((NPAD, FH), jnp.float32),
        pltpu.VMEM_SHARED((NPAD, L), jnp.float32),
        pltpu.SemaphoreType.DMA,
        pltpu.SemaphoreType.DMA,
        pltpu.SemaphoreType.DMA,
        pltpu.SemaphoreType.DMA,
        pltpu.SemaphoreType.DMA,
        pltpu.SemaphoreType.DMA,
        pltpu.SemaphoreType.DMA,
        pltpu.SemaphoreType.DMA,
        pltpu.SemaphoreType.DMA,
        pltpu.SemaphoreType.DMA,
        pltpu.SemaphoreType.DMA,
        pltpu.SemaphoreType.DMA,
    ],
)
def _edge_call(rowi_hbm, coli_hbm, a_hbm, b_hbm, c_hbm, w_hbm,
               hs_out, dg_out, s_out,
               rowb_v, colb_v, row2b_v, col2b_v, rscb_v, arb_v, bcb_v, cb_v,
               hb_v, sb_v, ones_v, sacc_v, w_v, hsh, dsh,
               semga0, semga1, semgb0, semgb1, semgc0, semgc1,
               semh0, semh1, semd0, semd1, sems0, sems1):
    cid = lax.axis_index("c")
    sid = lax.axis_index("s")

    pltpu.sync_copy(w_hbm.at[cid], w_v)

    zero16 = jnp.zeros((L,), jnp.float32)
    idx16 = lax.iota(jnp.int32, L)
    onevec = jnp.where(idx16 == 0, 1.0, 0.0).astype(jnp.float32)
    semga = [semga0, semga1]
    semgb = [semgb0, semgb1]
    semgc = [semgc0, semgc1]
    semh = [semh0, semh1]
    semd = [semd0, semd1]
    sems = [sems0, sems1]

    def _zrow(j, carry):
        for kk in range(FH // L):
            hb_v[0, j, pl.ds(kk * L, L)] = zero16
        ones_v[j, :] = zero16
        return carry

    lax.fori_loop(0, EB, _zrow, 0)

    def _zcp(t, carry):
        r0 = sid * RPT + t * EB
        pltpu.async_copy(hb_v.at[0], hsh.at[pl.ds(r0, EB)], semga0)
        pltpu.async_copy(ones_v, dsh.at[pl.ds(r0, EB)], semgb0)
        return carry

    lax.fori_loop(0, RPT // EB, _zcp, 0)

    def _zwait(t, carry):
        r0 = sid * RPT + t * EB
        pltpu.make_async_copy(
            hb_v.at[0], hsh.at[pl.ds(r0, EB)], semga0).wait()
        pltpu.make_async_copy(
            ones_v, dsh.at[pl.ds(r0, EB)], semgb0).wait()
        return carry

    lax.fori_loop(0, RPT // EB, _zwait, 0)

    def _orow(j, carry):
        ones_v[j, :] = onevec
        return carry

    lax.fori_loop(0, EB, _orow, 0)

    plsc.subcore_barrier()

    wregs = [w_v[pl.ds(kk * L, L)] for kk in range(FH // L)]
    roff = cid * N
    ebase = sid * EPT

    def _issue(p, bi):
        base = ebase + bi * EB
        pltpu.sync_copy(rowi_hbm.at[pl.ds(base, EB)], rowb_v.at[p])
        pltpu.sync_copy(coli_hbm.at[pl.ds(base, EB)], colb_v.at[p])
        for g in range(EB // L):
            s = pl.ds(g * L, L)
            row2b_v[p, s] = rowb_v[p, s] + roff
            col2b_v[p, s] = colb_v[p, s] + roff
        pltpu.async_copy(a_hbm.at[row2b_v.at[p]], arb_v.at[p], semga[p])
        pltpu.async_copy(b_hbm.at[col2b_v.at[p]], bcb_v.at[p], semgb[p])
        pltpu.async_copy(c_hbm.at[pl.ds(cid * E + base, EB)], cb_v.at[p],
                         semgc[p])

    def _waitg(p, bi):
        base = ebase + bi * EB
        pltpu.make_async_copy(
            a_hbm.at[row2b_v.at[p]], arb_v.at[p], semga[p]).wait()
        pltpu.make_async_copy(
            b_hbm.at[col2b_v.at[p]], bcb_v.at[p], semgb[p]).wait()
        pltpu.make_async_copy(
            c_hbm.at[pl.ds(cid * E + base, EB)], cb_v.at[p], semgc[p]).wait()

    def _compute(p, bi):
        def _row(j, rc):
            acc = zero16
            for kk in range(FH // L):
                s = pl.ds(kk * L, L)
                hv = jnp.maximum(
                    arb_v[p, j, s] + bcb_v[p, j, s] + cb_v[p, j, s], 0.0)
                hb_v[p, j, s] = hv
                acc = acc + hv * wregs[kk]
            sacc_v[j, :] = acc
            return rc

        lax.fori_loop(0, EB, _row, 0)

        for g in range(EB // L):
            eidx = idx16 + g * L
            sv = zero16
            for l in range(L):
                sv = sv + plsc.load_gather(
                    sacc_v, [eidx, jnp.full((L,), l, jnp.int32)])
            sb_v[p, pl.ds(g * L, L)] = sv
            s = pl.ds(g * L, L)
            rscb_v[p, s] = rowb_v[p, s]

    def _scatter(p, bi):
        base = ebase + bi * EB
        pltpu.async_copy(hb_v.at[p], hsh.at[rscb_v.at[p]], semh[p],
                         add=True)

        @pl.when(cid == 0)
        def _deg():
            pltpu.async_copy(ones_v, dsh.at[rscb_v.at[p]], semd[p],
                             add=True)

        pltpu.async_copy(sb_v.at[p], s_out.at[pl.ds(cid * E + base, EB)],
                         sems[p])

    def _drain_sc(p, bi_prev):
        base = ebase + bi_prev * EB
        pltpu.make_async_copy(
            hb_v.at[p], hsh.at[rscb_v.at[p]], semh[p]).wait()

        @pl.when(cid == 0)
        def _deg():
            pltpu.make_async_copy(
                ones_v, dsh.at[rscb_v.at[p]], semd[p]).wait()

        pltpu.make_async_copy(
            sb_v.at[p], s_out.at[pl.ds(cid * E + base, EB)], sems[p]).wait()

    NHALF = NB1 // 2
    _issue(0, 0)
    _issue(1, 1)
    _waitg(0, 0)
    _compute(0, 0)
    _scatter(0, 0)
    _issue(0, 2)
    _waitg(1, 1)
    _compute(1, 1)
    _scatter(1, 1)
    _issue(1, 3)

    def _pair(k, carry):
        for p in range(2):
            bi = 2 * k + p
            _drain_sc(p, bi - 2)
            _waitg(p, bi)
            _compute(p, bi)
            _scatter(p, bi)

            @pl.when(k < NHALF - 1)
            def _next():
                _issue(p, bi + 2)

        return carry

    lax.fori_loop(1, NHALF, _pair, 0)
    _drain_sc(0, NB1 - 2)
    _drain_sc(1, NB1 - 1)

    plsc.subcore_barrier()

    r0 = sid * RPT

    @pl.when(sid < NS - 1)
    def _drain_full():
        pltpu.sync_copy(hsh.at[pl.ds(r0, RPT)], hs_out.at[cid, pl.ds(r0, RPT)])

        @pl.when(cid == 0)
        def _():
            pltpu.sync_copy(dsh.at[pl.ds(r0, RPT)], dg_out.at[pl.ds(r0, RPT)])

    @pl.when(sid == NS - 1)
    def _drain_last():
        pltpu.sync_copy(hsh.at[pl.ds(r0, LAST_ROWS)],
                        hs_out.at[cid, pl.ds(r0, LAST_ROWS)])

        @pl.when(cid == 0)
        def _():
            pltpu.sync_copy(dsh.at[pl.ds(r0, LAST_ROWS)],
                            dg_out.at[pl.ds(r0, LAST_ROWS)])


@functools.partial(
    pl.kernel,
    out_type=[
        jax.ShapeDtypeStruct((NC, N, L), jnp.float32),
        jax.ShapeDtypeStruct((NC, N, L), jnp.float32),
    ],
    mesh=_mesh,
    compiler_params=_sc_params,
    scratch_types=[
        pltpu.VMEM((2, EB), jnp.int32),
        pltpu.VMEM((2, EB), jnp.int32),
        pltpu.VMEM((2, EB), jnp.int32),
        pltpu.VMEM((2, EB), jnp.int32),
        pltpu.VMEM((2, EB), jnp.float32),
        pltpu.VMEM((2, EB), jnp.float32),
        pltpu.VMEM((2, EB, L), jnp.float32),
        pltpu.VMEM((L,), jnp.float32),
        pltpu.VMEM_SHARED((NPAD, L), jnp.float32),
        pltpu.VMEM_SHARED((NPAD, L), jnp.float32),
        pltpu.SemaphoreType.DMA,
        pltpu.SemaphoreType.DMA,
        pltpu.SemaphoreType.DMA,
        pltpu.SemaphoreType.DMA,
        pltpu.SemaphoreType.DMA,
        pltpu.SemaphoreType.DMA,
        pltpu.SemaphoreType.DMA,
        pltpu.SemaphoreType.DMA,
    ],
)
def _coord_call(rowi_hbm, coli_hbm, s_hbm, k_hbm, cr_out, cc_out,
                rowb_v, colb_v, rscb_v, cscb_v, s0b_v, s1b_v, cbufb_v, k_v,
                crsh, ccsh,
                semi0, semi1, sems0, sems1, semr0, semr1, semc0, semc1):
    cid = lax.axis_index("c")
    sid = lax.axis_index("s")
    wid = sid * NC + cid

    pltpu.sync_copy(k_hbm, k_v)
    zero16 = jnp.zeros((L,), jnp.float32)
    idx16 = lax.iota(jnp.int32, L)
    zeroi = jnp.zeros((L,), jnp.int32)
    semi = [semi0, semi1]
    semsv = [sems0, sems1]
    semr = [semr0, semr1]
    semc = [semc0, semc1]

    def _zrow(j, carry):
        for p in range(2):
            cbufb_v[p, j, :] = zero16
        return carry

    lax.fori_loop(0, EB, _zrow, 0)

    def _zcp(t, carry):
        r0 = sid * RPT + t * EB
        pltpu.async_copy(cbufb_v.at[0], crsh.at[pl.ds(r0, EB)], semi0)
        pltpu.async_copy(cbufb_v.at[0], ccsh.at[pl.ds(r0, EB)], sems0)
        return carry

    lax.fori_loop(0, RPT // EB, _zcp, 0)

    def _zwait(t, carry):
        r0 = sid * RPT + t * EB
        pltpu.make_async_copy(
            cbufb_v.at[0], crsh.at[pl.ds(r0, EB)], semi0).wait()
        pltpu.make_async_copy(
            cbufb_v.at[0], ccsh.at[pl.ds(r0, EB)], sems0).wait()
        return carry

    lax.fori_loop(0, RPT // EB, _zwait, 0)

    plsc.subcore_barrier()

    kvec = k_v[:]
    b0 = kvec[0]
    cw2s = kvec[1]
    cb2s = kvec[2]
    ebase = wid * EPW

    def _issue(p, bi):
        base = ebase + bi * EB
        pltpu.async_copy(rowi_hbm.at[pl.ds(base, EB)], rowb_v.at[p], semi[p])
        pltpu.async_copy(coli_hbm.at[pl.ds(base, EB)], colb_v.at[p], semi[p])
        pltpu.async_copy(s_hbm.at[pl.ds(base, EB)], s0b_v.at[p], semsv[p])
        pltpu.async_copy(s_hbm.at[pl.ds(E + base, EB)], s1b_v.at[p], semsv[p])

    def _waitin(p, bi):
        base = ebase + bi * EB
        pltpu.make_async_copy(
            rowi_hbm.at[pl.ds(base, EB)], rowb_v.at[p], semi[p]).wait()
        pltpu.make_async_copy(
            coli_hbm.at[pl.ds(base, EB)], colb_v.at[p], semi[p]).wait()
        pltpu.make_async_copy(
            s_hbm.at[pl.ds(base, EB)], s0b_v.at[p], semsv[p]).wait()
        pltpu.make_async_copy(
            s_hbm.at[pl.ds(E + base, EB)], s1b_v.at[p], semsv[p]).wait()

    def _compute(p):
        for g in range(EB // L):
            s = pl.ds(g * L, L)
            sv = s0b_v[p, s] + s1b_v[p, s]
            cvec = jnp.maximum(sv + b0, 0.0) * cw2s + cb2s
            plsc.store_scatter(cbufb_v.at[p], [idx16 + g * L, zeroi], cvec)
            rscb_v[p, s] = rowb_v[p, s]
            cscb_v[p, s] = colb_v[p, s]

    def _scatter(p):
        pltpu.async_copy(cbufb_v.at[p], crsh.at[rscb_v.at[p]], semr[p],
                         add=True)
        pltpu.async_copy(cbufb_v.at[p], ccsh.at[cscb_v.at[p]], semc[p],
                         add=True)

    def _drain(p):
        pltpu.make_async_copy(
            cbufb_v.at[p], crsh.at[rscb_v.at[p]], semr[p]).wait()
        pltpu.make_async_copy(
            cbufb_v.at[p], ccsh.at[cscb_v.at[p]], semc[p]).wait()

    NHALF2 = NB2 // 2
    _issue(0, 0)
    _issue(1, 1)
    _waitin(0, 0)
    _compute(0)
    _scatter(0)
    _issue(0, 2)
    _waitin(1, 1)
    _compute(1)
    _scatter(1)
    _issue(1, 3)

    def _pair(k, carry):
        for p in range(2):
            bi = 2 * k + p
            _drain(p)
            _waitin(p, bi)
            _compute(p)
            _scatter(p)

            @pl.when(bi + 2 < NB2)
            def _next():
                _issue(p, bi + 2)

        return carry

    lax.fori_loop(1, NHALF2, _pair, 0)
    if NB2 % 2 == 1:
        _drain(0)
        _waitin(0, NB2 - 1)
        _compute(0)
        _scatter(0)
    _drain(0)
    _drain(1)

    plsc.subcore_barrier()

    r0 = sid * RPT

    @pl.when(sid < NS - 1)
    def _drain_full():
        pltpu.sync_copy(crsh.at[pl.ds(r0, RPT)], cr_out.at[cid, pl.ds(r0, RPT)])
        pltpu.sync_copy(ccsh.at[pl.ds(r0, RPT)], cc_out.at[cid, pl.ds(r0, RPT)])

    @pl.when(sid == NS - 1)
    def _drain_last():
        pltpu.sync_copy(crsh.at[pl.ds(r0, LAST_ROWS)],
                        cr_out.at[cid, pl.ds(r0, LAST_ROWS)])
        pltpu.sync_copy(ccsh.at[pl.ds(r0, LAST_ROWS)],
                        cc_out.at[cid, pl.ds(r0, LAST_ROWS)])


def kernel(x, edge_index, edge_attr, pos,
           ew1, eb1, ew2, eb2,
           nw1, nb1, nw2, nb2,
           cw1, cb1, cw2, cb2):
    rowi = edge_index[0].astype(jnp.int32)
    coli = edge_index[1].astype(jnp.int32)

    a_st, b_st, c_st = _abc_call(
        x, ew1[:D, :FH], ew1[:D, FH:], ew1[D:2 * D, :FH], ew1[D:2 * D, FH:],
        edge_attr, ew1[2 * D:, :FH], ew1[2 * D:, FH:],
        eb1[:FH].reshape(1, FH), eb1[FH:].reshape(1, FH))

    w = (ew2 @ cw1)[:, 0]
    b0 = eb2 @ cw1[:, 0] + cb1[0]
    consts = jnp.concatenate([
        jnp.reshape(b0, (1,)), jnp.reshape(cw2, (1,)), jnp.reshape(cb2, (1,)),
        jnp.zeros((L - 3,), jnp.float32),
    ])

    hs, dg, s_parts = _edge_call(
        rowi, coli,
        a_st.reshape(NC * N, FH), b_st.reshape(NC * N, FH),
        c_st.reshape(NC * E, FH), w.reshape(NC, FH))

    crow, ccol = _coord_call(rowi, coli, s_parts, consts)

    node_features = _node_call(
        x, hs, hs, dg, ew2, eb2.reshape(1, F),
        nw1[:D], nw1[D:], nb1.reshape(1, F), nw2, nb2.reshape(1, F))
    co_parts = _coordred_call(crow, crow, ccol, ccol, pos)

    coord = co_parts.reshape(N // NODE_BLK, 3).sum(axis=0)
    pos_out = pos + coord[None, :]
    return node_features, pos_out
